# Initial kernel scaffold; baseline (speedup 1.0000x reference)
import jax
import jax.numpy as jnp
from jax.experimental import pallas as pl


def kernel(x, edge_index, batch, W1, b1, W2, b2, W3, b3, W4, b4):
    raise NotImplementedError("write your pallas kernel here")


# speedup vs baseline: 20.9761x; 20.9761x over previous
import functools

import jax
import jax.numpy as jnp
from jax import lax
from jax.experimental import pallas as pl
from jax.experimental.pallas import tpu as pltpu
from jax.experimental.pallas import tpu_sc as plsc

N = 10000
E = 160000
DIN = 256
DH = 512
DZ = 128
G = 64
NPAD = 10240
NW = 32
EW = E // NW
EC = E // 16
CH = 128
NCH_W = (EW + CH - 1) // CH
NCH_C = (EC + CH - 1) // CH
ROWS_PER_SUB = NPAD // 16

_mesh = plsc.VectorSubcoreMesh(core_axis_name="c", subcore_axis_name="s")


def _zero_rows(zeros_hbm, sp_ref, s):
    base = s * ROWS_PER_SUB
    for k in range(ROWS_PER_SUB // 128):
        pltpu.sync_copy(zeros_hbm, sp_ref.at[pl.ds(base + k * 128, 128)])


@functools.partial(
    pl.kernel,
    out_type=jax.ShapeDtypeStruct((2 * NPAD, 16), jnp.float32),
    mesh=_mesh,
    scratch_types=[
        pltpu.VMEM((NCH_W, CH), jnp.int32),
        pltpu.VMEM((CH, 16), jnp.float32),
        pltpu.VMEM_SHARED((NPAD, 16), jnp.float32),
    ],
    compiler_params=pltpu.CompilerParams(
        needs_layout_passes=False, use_tc_tiling_on_sc=False),
)
def _deg_kernel(dst_hbm, onehot_hbm, zeros16_hbm, out_hbm, d2, oh, deg_sp):
    c = lax.axis_index("c")
    s = lax.axis_index("s")
    _zero_rows(zeros16_hbm, deg_sp, s)
    pltpu.sync_copy(onehot_hbm, oh)
    pltpu.sync_copy(dst_hbm.at[c * 16 + s], d2)
    plsc.subcore_barrier()

    def body(k, carry):
        pltpu.sync_copy(oh, deg_sp.at[d2.at[k]], add=True)
        return carry

    lax.fori_loop(0, NCH_W, body, 0)
    plsc.subcore_barrier()
    pltpu.sync_copy(deg_sp.at[pl.ds(s * ROWS_PER_SUB, ROWS_PER_SUB)],
                    out_hbm.at[pl.ds(c * NPAD + s * ROWS_PER_SUB,
                                     ROWS_PER_SUB)])


_QSHAPE = jax.ShapeDtypeStruct((NPAD, G), jnp.float32)


@functools.partial(
    pl.kernel,
    out_type=(
        _QSHAPE, _QSHAPE, _QSHAPE, _QSHAPE,
        _QSHAPE, _QSHAPE,
    ),
    mesh=_mesh,
    scratch_types=[
        pltpu.VMEM((CH * NCH_C,), jnp.int32),
        pltpu.VMEM((NCH_C, CH), jnp.int32),
        pltpu.VMEM((CH, G), jnp.float32),
        pltpu.VMEM((NCH_W, CH), jnp.int32),
        pltpu.VMEM((CH * NCH_W,), jnp.int32),
        pltpu.VMEM((N,), jnp.int32),
        pltpu.VMEM((N,), jnp.float32),
        pltpu.VMEM((CH, G), jnp.float32),
        pltpu.VMEM_SHARED((NPAD, G), jnp.float32),
        pltpu.SemaphoreType.DMA,
    ],
    compiler_params=pltpu.CompilerParams(
        needs_layout_passes=False, use_tc_tiling_on_sc=False),
)
def _scatter_kernel(xs0, xs1, xs2, xs3, srcy, dsty, sct0, sct1, dct0, dct1,
                    batch_hbm, dinv_hbm, zeros_hbm,
                    y0_out, y1_out, y2_out, y3_out, ct0_out, ct1_out,
                    sbuf, d2, gbuf, srows, dbuf, btab, dtab, oh, acc_sp,
                    sem):
    c = lax.axis_index("c")
    s = lax.axis_index("s")
    iota = lax.iota(jnp.int32, 16)
    rows = pl.ds(s * ROWS_PER_SUB, ROWS_PER_SUB)

    pltpu.sync_copy(zeros_hbm, oh)
    pltpu.sync_copy(srcy.at[s], sbuf)
    pltpu.sync_copy(dsty.at[s], d2)
    pltpu.sync_copy(batch_hbm, btab)
    pltpu.sync_copy(dinv_hbm, dtab)

    def y_scatter(xs_ref):
        def body(j, carry):
            pltpu.async_copy(
                xs_ref.at[sbuf.at[pl.ds(j * CH, CH)]], gbuf, sem).wait()
            pltpu.sync_copy(gbuf, acc_sp.at[d2.at[j]], add=True)
            return carry
        lax.fori_loop(0, NCH_C, body, 0)

    def ct_scatter(sct, dct):
        pltpu.sync_copy(sct.at[s], srows)
        pltpu.sync_copy(dct.at[s], dbuf)

        def body(k, carry):
            for v in range(8):
                d = dbuf[pl.ds(k * CH + v * 16, 16)]
                gidx = plsc.load_gather(btab, [d])
                val = plsc.load_gather(dtab, [d])
                e = v * 16 + iota
                plsc.store_scatter(oh, [e, gidx], val)
            pltpu.sync_copy(oh, acc_sp.at[srows.at[k]], add=True)
            for v in range(8):
                d = dbuf[pl.ds(k * CH + v * 16, 16)]
                gidx = plsc.load_gather(btab, [d])
                e = v * 16 + iota
                plsc.store_scatter(oh, [e, gidx], jnp.zeros((16,), jnp.float32))
            return carry
        lax.fori_loop(0, NCH_W, body, 0)

    def acc_pass(scatter_fn, out_ref):
        _zero_rows(zeros_hbm, acc_sp, s)
        plsc.subcore_barrier()
        scatter_fn()
        plsc.subcore_barrier()
        pltpu.sync_copy(acc_sp.at[rows], out_ref.at[rows])
        plsc.subcore_barrier()

    @pl.when(c == 0)
    def _():
        acc_pass(lambda: y_scatter(xs0), y0_out)
        acc_pass(lambda: y_scatter(xs1), y1_out)
        acc_pass(lambda: ct_scatter(sct0, dct0), ct0_out)

    @pl.when(c == 1)
    def _():
        acc_pass(lambda: y_scatter(xs2), y2_out)
        acc_pass(lambda: y_scatter(xs3), y3_out)
        acc_pass(lambda: ct_scatter(sct1, dct1), ct1_out)


_BLK = 1000
_NBLK = N // _BLK


def _prep_body(ind_ref, x_ref, batch_ref, xs0_ref, xs1_ref, xs2_ref, xs3_ref,
               dinv_ref, cntinv_ref, cnt_acc):
    i = pl.program_id(0)
    deg = ind_ref[:, 0:1] + ind_ref[:, 1:2] + 1.0
    dinv = lax.rsqrt(deg)
    dinv_ref[...] = dinv
    xs = x_ref[...] * dinv
    xs0_ref[...] = xs[:, 0:64]
    xs1_ref[...] = xs[:, 64:128]
    xs2_ref[...] = xs[:, 128:192]
    xs3_ref[...] = xs[:, 192:256]
    onehot = (batch_ref[...] ==
              lax.broadcasted_iota(jnp.int32, (_BLK, G), 1)).astype(jnp.float32)

    @pl.when(i == 0)
    def _():
        cnt_acc[...] = jnp.zeros_like(cnt_acc)

    cnt_acc[...] += jnp.sum(onehot, axis=0, keepdims=True)

    @pl.when(i == _NBLK - 1)
    def _():
        cntinv_ref[...] = 1.0 / jnp.maximum(cnt_acc[...], 1.0)


_prep_call = pl.pallas_call(
    _prep_body,
    grid=(_NBLK,),
    in_specs=[
        pl.BlockSpec((_BLK, 2), lambda i: (i, 0)),
        pl.BlockSpec((_BLK, DIN), lambda i: (i, 0)),
        pl.BlockSpec((_BLK, 1), lambda i: (i, 0)),
    ],
    out_specs=[
        pl.BlockSpec((_BLK, G), lambda i: (i, 0)),
        pl.BlockSpec((_BLK, G), lambda i: (i, 0)),
        pl.BlockSpec((_BLK, G), lambda i: (i, 0)),
        pl.BlockSpec((_BLK, G), lambda i: (i, 0)),
        pl.BlockSpec((_BLK, 1), lambda i: (i, 0)),
        pl.BlockSpec((1, G), lambda i: (0, 0)),
    ],
    out_shape=[
        jax.ShapeDtypeStruct((N, G), jnp.float32),
        jax.ShapeDtypeStruct((N, G), jnp.float32),
        jax.ShapeDtypeStruct((N, G), jnp.float32),
        jax.ShapeDtypeStruct((N, G), jnp.float32),
        jax.ShapeDtypeStruct((N, 1), jnp.float32),
        jax.ShapeDtypeStruct((1, G), jnp.float32),
    ],
    scratch_shapes=[pltpu.VMEM((1, G), jnp.float32)],
    compiler_params=pltpu.CompilerParams(
        dimension_semantics=("arbitrary",)),
)


def _dense_body(x_ref, y0_ref, y1_ref, y2_ref, y3_ref, ct0_ref, ct1_ref,
                dinv_ref, batch_ref,
                cntinv_ref, w1_ref, b1_ref, w3_ref, b3_ref,
                w2_ref, b2_ref, w4_ref, b4_ref,
                zm_ref, zl_ref, accm, accl):
    i = pl.program_id(0)
    dinv = dinv_ref[...]
    dinv2 = dinv * dinv
    y_acc = jnp.concatenate(
        [y0_ref[...], y1_ref[...], y2_ref[...], y3_ref[...]], axis=1)
    y = dinv * y_acc + dinv2 * x_ref[...]
    hm = jnp.maximum(
        jnp.dot(y, w1_ref[...], preferred_element_type=jnp.float32)
        + b1_ref[...], 0.0)
    hl = jnp.maximum(
        jnp.dot(y, w3_ref[...], preferred_element_type=jnp.float32)
        + b3_ref[...], 0.0)
    onehot = (batch_ref[...] ==
              lax.broadcasted_iota(jnp.int32, (_BLK, G), 1)).astype(jnp.float32)
    ctf = dinv * (ct0_ref[...] + ct1_ref[...]) + dinv2 * onehot
    dn = (((0,), (0,)), ((), ()))

    @pl.when(i == 0)
    def _():
        accm[...] = jnp.zeros_like(accm)
        accl[...] = jnp.zeros_like(accl)

    accm[...] += lax.dot_general(ctf, hm, dimension_numbers=dn,
                                 preferred_element_type=jnp.float32)
    accl[...] += lax.dot_general(ctf, hl, dimension_numbers=dn,
                                 preferred_element_type=jnp.float32)

    @pl.when(i == _NBLK - 1)
    def _():
        cntinv = cntinv_ref[...]
        zm_ref[...] = jnp.dot(accm[...] * cntinv, w2_ref[...],
                              preferred_element_type=jnp.float32) + b2_ref[...]
        zl_ref[...] = jnp.dot(accl[...] * cntinv, w4_ref[...],
                              preferred_element_type=jnp.float32) + b4_ref[...]


_dense_call = pl.pallas_call(
    _dense_body,
    grid=(_NBLK,),
    in_specs=[
        pl.BlockSpec((_BLK, DIN), lambda i: (i, 0)),
        pl.BlockSpec((_BLK, G), lambda i: (i, 0)),
        pl.BlockSpec((_BLK, G), lambda i: (i, 0)),
        pl.BlockSpec((_BLK, G), lambda i: (i, 0)),
        pl.BlockSpec((_BLK, G), lambda i: (i, 0)),
        pl.BlockSpec((_BLK, G), lambda i: (i, 0)),
        pl.BlockSpec((_BLK, G), lambda i: (i, 0)),
        pl.BlockSpec((_BLK, 1), lambda i: (i, 0)),
        pl.BlockSpec((_BLK, 1), lambda i: (i, 0)),
        pl.BlockSpec((G, 1), lambda i: (0, 0)),
        pl.BlockSpec((DIN, DH), lambda i: (0, 0)),
        pl.BlockSpec((1, DH), lambda i: (0, 0)),
        pl.BlockSpec((DIN, DH), lambda i: (0, 0)),
        pl.BlockSpec((1, DH), lambda i: (0, 0)),
        pl.BlockSpec((DH, DZ), lambda i: (0, 0)),
        pl.BlockSpec((1, DZ), lambda i: (0, 0)),
        pl.BlockSpec((DH, DZ), lambda i: (0, 0)),
        pl.BlockSpec((1, DZ), lambda i: (0, 0)),
    ],
    out_specs=[
        pl.BlockSpec((G, DZ), lambda i: (0, 0)),
        pl.BlockSpec((G, DZ), lambda i: (0, 0)),
    ],
    out_shape=[
        jax.ShapeDtypeStruct((G, DZ), jnp.float32),
        jax.ShapeDtypeStruct((G, DZ), jnp.float32),
    ],
    scratch_shapes=[
        pltpu.VMEM((G, DH), jnp.float32),
        pltpu.VMEM((G, DH), jnp.float32),
    ],
    compiler_params=pltpu.CompilerParams(
        dimension_semantics=("arbitrary",)),
)


def _pad_ids(ids, per_worker, nworkers, pad_base):
    padded = ((per_worker + CH - 1) // CH) * CH
    npad = padded - per_worker
    padv = pad_base + (jnp.arange(npad, dtype=jnp.int32) % 16)
    padv = jnp.broadcast_to(padv, (nworkers, npad))
    return jnp.concatenate([ids.reshape(nworkers, per_worker), padv], axis=1)


def kernel(x, edge_index, batch, W1, b1, W2, b2, W3, b3, W4, b4):
    src = edge_index[0].astype(jnp.int32)
    dst = edge_index[1].astype(jnp.int32)
    batch = batch.astype(jnp.int32)

    dst_deg = _pad_ids(dst, EW, NW, N).reshape(NW, NCH_W, CH)
    srcy = _pad_ids(src, EC, 16, 0)
    dsty = _pad_ids(dst, EC, 16, N).reshape(16, NCH_C, CH)
    sct = _pad_ids(src, EW, NW, N).reshape(2, 16, NCH_W, CH)
    dct = _pad_ids(dst, EW, NW, 0).reshape(2, 16, NCH_W * CH)

    zeros64 = jnp.zeros((128, G), jnp.float32)
    zeros16 = jnp.zeros((128, 16), jnp.float32)
    onehot16 = jnp.zeros((CH, 16), jnp.float32).at[:, 0].set(1.0)

    deg_raw = _deg_kernel(dst_deg, onehot16, zeros16)
    ind = deg_raw.reshape(2, NPAD, 16)[:, :N, 0].T

    xs0, xs1, xs2, xs3, dinv, cntinv = _prep_call(ind, x, batch.reshape(N, 1))

    y0p, y1p, y2p, y3p, ct0p, ct1p = _scatter_kernel(
        xs0, xs1, xs2, xs3, srcy, dsty, sct[0], sct[1], dct[0], dct[1],
        batch, dinv.reshape(N), zeros64)

    z_mean, z_logvar = _dense_call(
        x, y0p[:N], y1p[:N], y2p[:N], y3p[:N], ct0p[:N], ct1p[:N], dinv,
        batch.reshape(N, 1),
        cntinv.reshape(G, 1),
        W1, b1.reshape(1, DH), W3, b3.reshape(1, DH),
        W2, b2.reshape(1, DZ), W4, b4.reshape(1, DZ))
    return (z_mean, z_logvar)


# speedup vs baseline: 23.6781x; 1.1288x over previous
import functools

import jax
import jax.numpy as jnp
from jax import lax
from jax.experimental import pallas as pl
from jax.experimental.pallas import tpu as pltpu
from jax.experimental.pallas import tpu_sc as plsc

N = 10000
E = 160000
DIN = 256
DH = 512
DZ = 128
G = 64
NPAD = 10240
NW = 32
EW = E // NW
EC = E // 16
CH = 128
NCH_W = 40
NCH_C = 80
NBUF = 4
ROWS_PER_SUB = NPAD // 16

_mesh = plsc.VectorSubcoreMesh(core_axis_name="c", subcore_axis_name="s")


def _zero_rows(zeros_hbm, sp_ref, s):
    base = s * ROWS_PER_SUB
    for k in range(ROWS_PER_SUB // 128):
        pltpu.sync_copy(zeros_hbm, sp_ref.at[pl.ds(base + k * 128, 128)])


@functools.partial(
    pl.kernel,
    out_type=jax.ShapeDtypeStruct((2 * NPAD, 16), jnp.float32),
    mesh=_mesh,
    scratch_types=[
        pltpu.VMEM((NCH_W, CH), jnp.int32),
        pltpu.VMEM((CH, 16), jnp.float32),
        pltpu.VMEM_SHARED((NPAD, 16), jnp.float32),
    ],
    compiler_params=pltpu.CompilerParams(
        needs_layout_passes=False, use_tc_tiling_on_sc=False),
)
def _deg_kernel(dst_hbm, onehot_hbm, zeros16_hbm, out_hbm, d2, oh, deg_sp):
    c = lax.axis_index("c")
    s = lax.axis_index("s")
    _zero_rows(zeros16_hbm, deg_sp, s)
    pltpu.sync_copy(onehot_hbm, oh)
    pltpu.sync_copy(dst_hbm.at[c * 16 + s], d2)
    plsc.subcore_barrier()

    def body(k, carry):
        pltpu.sync_copy(oh, deg_sp.at[d2.at[k]], add=True)
        return carry

    lax.fori_loop(0, NCH_W, body, 0)
    plsc.subcore_barrier()
    pltpu.sync_copy(deg_sp.at[pl.ds(s * ROWS_PER_SUB, ROWS_PER_SUB)],
                    out_hbm.at[pl.ds(c * NPAD + s * ROWS_PER_SUB,
                                     ROWS_PER_SUB)])


_QSHAPE = jax.ShapeDtypeStruct((NPAD, G), jnp.float32)


@functools.partial(
    pl.kernel,
    out_type=(
        _QSHAPE, _QSHAPE, _QSHAPE, _QSHAPE,
        _QSHAPE, _QSHAPE,
    ),
    mesh=_mesh,
    scratch_types=[
        pltpu.VMEM((CH * NCH_C,), jnp.int32),
        pltpu.VMEM((NCH_C, CH), jnp.int32),
        pltpu.VMEM((CH * NCH_W,), jnp.int32),
        pltpu.VMEM((NCH_W, CH), jnp.int32),
        [pltpu.VMEM((CH, G), jnp.float32) for _ in range(NBUF)],
        pltpu.VMEM_SHARED((NPAD, G), jnp.float32),
        pltpu.SemaphoreType.DMA,
        pltpu.SemaphoreType.DMA,
    ],
    compiler_params=pltpu.CompilerParams(
        needs_layout_passes=False, use_tc_tiling_on_sc=False),
)
def _scatter_kernel(xs0, xs1, xs2, xs3, oht, srcy, dsty, sct0, sct1,
                    dct0, dct1, zeros_hbm,
                    y0_out, y1_out, y2_out, y3_out, ct0_out, ct1_out,
                    gidx1, srow2, cidx1, crow2, ring, acc_sp, gsem, ssem):
    c = lax.axis_index("c")
    s = lax.axis_index("s")
    rows = pl.ds(s * ROWS_PER_SUB, ROWS_PER_SUB)

    pltpu.sync_copy(srcy.at[s], gidx1)
    pltpu.sync_copy(dsty.at[s], srow2)

    def pipe_scatter(tab_ref, idx1, row2, nch):
        half = NBUF // 2
        nstep = nch // half

        def step(t, carry):
            def run_set(base):
                bufs = [ring[base + b] for b in range(half)]

                @pl.when(t >= 2)
                def _():
                    for b in range(half):
                        pltpu.make_async_copy(zeros_hbm, bufs[b], ssem).wait()

                hs = []
                for b in range(half):
                    j = t * half + b
                    hs.append(pltpu.async_copy(
                        tab_ref.at[idx1.at[pl.ds(j * CH, CH)]],
                        bufs[b], gsem))
                for h in hs:
                    h.wait()
                for b in range(half):
                    j = t * half + b
                    pltpu.async_copy(bufs[b], acc_sp.at[row2.at[j]],
                                     ssem, add=True)

            @pl.when(t % 2 == 0)
            def _():
                run_set(0)

            @pl.when(t % 2 == 1)
            def _():
                run_set(half)

            return carry

        lax.fori_loop(0, nstep, step, 0)
        for b in range(NBUF):
            pltpu.make_async_copy(zeros_hbm, ring[b], ssem).wait()

    def acc_pass(scatter_fn, out_ref):
        _zero_rows(zeros_hbm, acc_sp, s)
        plsc.subcore_barrier()
        scatter_fn()
        plsc.subcore_barrier()
        pltpu.sync_copy(acc_sp.at[rows], out_ref.at[rows])
        plsc.subcore_barrier()

    def ct_pass(sct, dct, ct_out):
        pltpu.sync_copy(dct.at[s], cidx1)
        pltpu.sync_copy(sct.at[s], crow2)
        acc_pass(lambda: pipe_scatter(oht, cidx1, crow2, NCH_W), ct_out)

    @pl.when(c == 0)
    def _():
        acc_pass(lambda: pipe_scatter(xs0, gidx1, srow2, NCH_C), y0_out)
        acc_pass(lambda: pipe_scatter(xs1, gidx1, srow2, NCH_C), y1_out)
        ct_pass(sct0, dct0, ct0_out)

    @pl.when(c == 1)
    def _():
        acc_pass(lambda: pipe_scatter(xs2, gidx1, srow2, NCH_C), y2_out)
        acc_pass(lambda: pipe_scatter(xs3, gidx1, srow2, NCH_C), y3_out)
        ct_pass(sct1, dct1, ct1_out)


_BLK = 1000
_NBLK = N // _BLK


def _prep_body(ind_ref, x_ref, batch_ref, xs0_ref, xs1_ref, xs2_ref, xs3_ref,
               dinv_ref, cntinv_ref, oht_ref, cnt_acc):
    i = pl.program_id(0)
    deg = ind_ref[:, 0:1] + ind_ref[:, 1:2] + 1.0
    dinv = lax.rsqrt(deg)
    dinv_ref[...] = dinv
    xs = x_ref[...] * dinv
    xs0_ref[...] = xs[:, 0:64]
    xs1_ref[...] = xs[:, 64:128]
    xs2_ref[...] = xs[:, 128:192]
    xs3_ref[...] = xs[:, 192:256]
    onehot = (batch_ref[...] ==
              lax.broadcasted_iota(jnp.int32, (_BLK, G), 1)).astype(jnp.float32)
    oht_ref[...] = onehot * dinv

    @pl.when(i == 0)
    def _():
        cnt_acc[...] = jnp.zeros_like(cnt_acc)

    cnt_acc[...] += jnp.sum(onehot, axis=0, keepdims=True)

    @pl.when(i == _NBLK - 1)
    def _():
        cntinv_ref[...] = 1.0 / jnp.maximum(cnt_acc[...], 1.0)


_prep_call = pl.pallas_call(
    _prep_body,
    grid=(_NBLK,),
    in_specs=[
        pl.BlockSpec((_BLK, 2), lambda i: (i, 0)),
        pl.BlockSpec((_BLK, DIN), lambda i: (i, 0)),
        pl.BlockSpec((_BLK, 1), lambda i: (i, 0)),
    ],
    out_specs=[
        pl.BlockSpec((_BLK, G), lambda i: (i, 0)),
        pl.BlockSpec((_BLK, G), lambda i: (i, 0)),
        pl.BlockSpec((_BLK, G), lambda i: (i, 0)),
        pl.BlockSpec((_BLK, G), lambda i: (i, 0)),
        pl.BlockSpec((_BLK, 1), lambda i: (i, 0)),
        pl.BlockSpec((1, G), lambda i: (0, 0)),
        pl.BlockSpec((_BLK, G), lambda i: (i, 0)),
    ],
    out_shape=[
        jax.ShapeDtypeStruct((N, G), jnp.float32),
        jax.ShapeDtypeStruct((N, G), jnp.float32),
        jax.ShapeDtypeStruct((N, G), jnp.float32),
        jax.ShapeDtypeStruct((N, G), jnp.float32),
        jax.ShapeDtypeStruct((N, 1), jnp.float32),
        jax.ShapeDtypeStruct((1, G), jnp.float32),
        jax.ShapeDtypeStruct((N, G), jnp.float32),
    ],
    scratch_shapes=[pltpu.VMEM((1, G), jnp.float32)],
    compiler_params=pltpu.CompilerParams(
        dimension_semantics=("arbitrary",)),
)


def _dense_body(x_ref, y0_ref, y1_ref, y2_ref, y3_ref, ct0_ref, ct1_ref,
                dinv_ref, batch_ref,
                cntinv_ref, w1_ref, b1_ref, w3_ref, b3_ref,
                w2_ref, b2_ref, w4_ref, b4_ref,
                zm_ref, zl_ref, accm, accl):
    i = pl.program_id(0)
    dinv = dinv_ref[...]
    dinv2 = dinv * dinv
    y_acc = jnp.concatenate(
        [y0_ref[...], y1_ref[...], y2_ref[...], y3_ref[...]], axis=1)
    y = dinv * y_acc + dinv2 * x_ref[...]
    hm = jnp.maximum(
        jnp.dot(y, w1_ref[...], preferred_element_type=jnp.float32)
        + b1_ref[...], 0.0)
    hl = jnp.maximum(
        jnp.dot(y, w3_ref[...], preferred_element_type=jnp.float32)
        + b3_ref[...], 0.0)
    onehot = (batch_ref[...] ==
              lax.broadcasted_iota(jnp.int32, (_BLK, G), 1)).astype(jnp.float32)
    ctf = dinv * (ct0_ref[...] + ct1_ref[...]) + dinv2 * onehot
    dn = (((0,), (0,)), ((), ()))

    @pl.when(i == 0)
    def _():
        accm[...] = jnp.zeros_like(accm)
        accl[...] = jnp.zeros_like(accl)

    accm[...] += lax.dot_general(ctf, hm, dimension_numbers=dn,
                                 preferred_element_type=jnp.float32)
    accl[...] += lax.dot_general(ctf, hl, dimension_numbers=dn,
                                 preferred_element_type=jnp.float32)

    @pl.when(i == _NBLK - 1)
    def _():
        cntinv = cntinv_ref[...]
        zm_ref[...] = jnp.dot(accm[...] * cntinv, w2_ref[...],
                              preferred_element_type=jnp.float32) + b2_ref[...]
        zl_ref[...] = jnp.dot(accl[...] * cntinv, w4_ref[...],
                              preferred_element_type=jnp.float32) + b4_ref[...]


_dense_call = pl.pallas_call(
    _dense_body,
    grid=(_NBLK,),
    in_specs=[
        pl.BlockSpec((_BLK, DIN), lambda i: (i, 0)),
        pl.BlockSpec((_BLK, G), lambda i: (i, 0)),
        pl.BlockSpec((_BLK, G), lambda i: (i, 0)),
        pl.BlockSpec((_BLK, G), lambda i: (i, 0)),
        pl.BlockSpec((_BLK, G), lambda i: (i, 0)),
        pl.BlockSpec((_BLK, G), lambda i: (i, 0)),
        pl.BlockSpec((_BLK, G), lambda i: (i, 0)),
        pl.BlockSpec((_BLK, 1), lambda i: (i, 0)),
        pl.BlockSpec((_BLK, 1), lambda i: (i, 0)),
        pl.BlockSpec((G, 1), lambda i: (0, 0)),
        pl.BlockSpec((DIN, DH), lambda i: (0, 0)),
        pl.BlockSpec((1, DH), lambda i: (0, 0)),
        pl.BlockSpec((DIN, DH), lambda i: (0, 0)),
        pl.BlockSpec((1, DH), lambda i: (0, 0)),
        pl.BlockSpec((DH, DZ), lambda i: (0, 0)),
        pl.BlockSpec((1, DZ), lambda i: (0, 0)),
        pl.BlockSpec((DH, DZ), lambda i: (0, 0)),
        pl.BlockSpec((1, DZ), lambda i: (0, 0)),
    ],
    out_specs=[
        pl.BlockSpec((G, DZ), lambda i: (0, 0)),
        pl.BlockSpec((G, DZ), lambda i: (0, 0)),
    ],
    out_shape=[
        jax.ShapeDtypeStruct((G, DZ), jnp.float32),
        jax.ShapeDtypeStruct((G, DZ), jnp.float32),
    ],
    scratch_shapes=[
        pltpu.VMEM((G, DH), jnp.float32),
        pltpu.VMEM((G, DH), jnp.float32),
    ],
    compiler_params=pltpu.CompilerParams(
        dimension_semantics=("arbitrary",)),
)


def _pad_ids(ids, per_worker, nworkers, pad_base, nch):
    padded = nch * CH
    npad = padded - per_worker
    padv = pad_base + (jnp.arange(npad, dtype=jnp.int32) % 16)
    padv = jnp.broadcast_to(padv, (nworkers, npad))
    return jnp.concatenate([ids.reshape(nworkers, per_worker), padv], axis=1)


def kernel(x, edge_index, batch, W1, b1, W2, b2, W3, b3, W4, b4):
    src = edge_index[0].astype(jnp.int32)
    dst = edge_index[1].astype(jnp.int32)
    batch = batch.astype(jnp.int32)

    dst_deg = _pad_ids(dst, EW, NW, N, NCH_W).reshape(NW, NCH_W, CH)
    srcy = _pad_ids(src, EC, 16, 0, NCH_C)
    dsty = _pad_ids(dst, EC, 16, N, NCH_C).reshape(16, NCH_C, CH)
    sct = _pad_ids(src, EW, NW, N, NCH_W).reshape(2, 16, NCH_W, CH)
    dct = _pad_ids(dst, EW, NW, 0, NCH_W).reshape(2, 16, NCH_W * CH)

    zeros64 = jnp.zeros((128, G), jnp.float32)
    zeros16 = jnp.zeros((128, 16), jnp.float32)
    onehot16 = jnp.zeros((CH, 16), jnp.float32).at[:, 0].set(1.0)

    deg_raw = _deg_kernel(dst_deg, onehot16, zeros16)
    ind = deg_raw.reshape(2, NPAD, 16)[:, :N, 0].T

    xs0, xs1, xs2, xs3, dinv, cntinv, oht = _prep_call(
        ind, x, batch.reshape(N, 1))

    y0p, y1p, y2p, y3p, ct0p, ct1p = _scatter_kernel(
        xs0, xs1, xs2, xs3, oht, srcy, dsty, sct[0], sct[1],
        dct[0], dct[1], zeros64)

    z_mean, z_logvar = _dense_call(
        x, y0p[:N], y1p[:N], y2p[:N], y3p[:N], ct0p[:N], ct1p[:N], dinv,
        batch.reshape(N, 1),
        cntinv.reshape(G, 1),
        W1, b1.reshape(1, DH), W3, b3.reshape(1, DH),
        W2, b2.reshape(1, DZ), W4, b4.reshape(1, DZ))
    return (z_mean, z_logvar)


# speedup vs baseline: 28.0969x; 1.1866x over previous
import functools

import jax
import jax.numpy as jnp
from jax import lax
from jax.experimental import pallas as pl
from jax.experimental.pallas import tpu as pltpu
from jax.experimental.pallas import tpu_sc as plsc

N = 10000
E = 160000
DIN = 256
DH = 512
DZ = 128
G = 64
NPAD = 10240
NW = 32
EW = E // NW
EC = E // 16
CH = 128
NCH_W = 40
NCH_C = 80
NBUF = 4
ROWS_PER_SUB = NPAD // 16

_mesh = plsc.VectorSubcoreMesh(core_axis_name="c", subcore_axis_name="s")


def _zero_rows(zeros_hbm, sp_ref, s):
    base = s * ROWS_PER_SUB
    for k in range(ROWS_PER_SUB // 128):
        pltpu.sync_copy(zeros_hbm, sp_ref.at[pl.ds(base + k * 128, 128)])


@functools.partial(
    pl.kernel,
    out_type=jax.ShapeDtypeStruct((2 * NPAD, 16), jnp.float32),
    mesh=_mesh,
    scratch_types=[
        pltpu.VMEM((NCH_W, CH), jnp.int32),
        pltpu.VMEM((CH, 16), jnp.float32),
        pltpu.VMEM_SHARED((NPAD, 16), jnp.float32),
    ],
    compiler_params=pltpu.CompilerParams(
        needs_layout_passes=False, use_tc_tiling_on_sc=False),
)
def _deg_kernel(dst_hbm, onehot_hbm, zeros16_hbm, out_hbm, d2, oh, deg_sp):
    c = lax.axis_index("c")
    s = lax.axis_index("s")
    _zero_rows(zeros16_hbm, deg_sp, s)
    pltpu.sync_copy(onehot_hbm, oh)
    pltpu.sync_copy(dst_hbm.at[c * 16 + s], d2)
    plsc.subcore_barrier()

    def body(k, carry):
        pltpu.sync_copy(oh, deg_sp.at[d2.at[k]], add=True)
        return carry

    lax.fori_loop(0, NCH_W, body, 0)
    plsc.subcore_barrier()
    pltpu.sync_copy(deg_sp.at[pl.ds(s * ROWS_PER_SUB, ROWS_PER_SUB)],
                    out_hbm.at[pl.ds(c * NPAD + s * ROWS_PER_SUB,
                                     ROWS_PER_SUB)])


_QSHAPE = jax.ShapeDtypeStruct((NPAD, G), jnp.float32)


@functools.partial(
    pl.kernel,
    out_type=(
        _QSHAPE, _QSHAPE, _QSHAPE, _QSHAPE,
        _QSHAPE, _QSHAPE,
    ),
    mesh=_mesh,
    scratch_types=[
        pltpu.VMEM((CH * NCH_C,), jnp.int32),
        pltpu.VMEM((NCH_C, CH), jnp.int32),
        pltpu.VMEM((CH * NCH_W,), jnp.int32),
        pltpu.VMEM((NCH_W, CH), jnp.int32),
        [pltpu.VMEM((CH, G), jnp.float32) for _ in range(NBUF)],
        pltpu.VMEM_SHARED((NPAD, G), jnp.float32),
        pltpu.SemaphoreType.DMA,
        pltpu.SemaphoreType.DMA,
    ],
    compiler_params=pltpu.CompilerParams(
        needs_layout_passes=False, use_tc_tiling_on_sc=False),
)
def _scatter_kernel(xs0, xs1, xs2, xs3, oht, srcy, dsty, sct0, sct1,
                    dct0, dct1, zeros_hbm,
                    y0_out, y1_out, y2_out, y3_out, ct0_out, ct1_out,
                    gidx1, srow2, cidx1, crow2, ring, acc_sp, gsem, ssem):
    c = lax.axis_index("c")
    s = lax.axis_index("s")
    rows = pl.ds(s * ROWS_PER_SUB, ROWS_PER_SUB)

    pltpu.sync_copy(srcy.at[s], gidx1)
    pltpu.sync_copy(dsty.at[s], srow2)

    def pipe_scatter(tab_ref, idx1, row2, nch):
        half = NBUF // 2
        nstep = nch // half

        def step(t, carry):
            def run_set(base):
                bufs = [ring[base + b] for b in range(half)]

                @pl.when(t >= 2)
                def _():
                    for b in range(half):
                        pltpu.make_async_copy(zeros_hbm, bufs[b], ssem).wait()

                hs = []
                for b in range(half):
                    j = t * half + b
                    hs.append(pltpu.async_copy(
                        tab_ref.at[idx1.at[pl.ds(j * CH, CH)]],
                        bufs[b], gsem))
                for h in hs:
                    h.wait()
                for b in range(half):
                    j = t * half + b
                    pltpu.async_copy(bufs[b], acc_sp.at[row2.at[j]],
                                     ssem, add=True)

            @pl.when(t % 2 == 0)
            def _():
                run_set(0)

            @pl.when(t % 2 == 1)
            def _():
                run_set(half)

            return carry

        lax.fori_loop(0, nstep, step, 0)
        for b in range(NBUF):
            pltpu.make_async_copy(zeros_hbm, ring[b], ssem).wait()

    def acc_pass(scatter_fn, out_ref):
        _zero_rows(zeros_hbm, acc_sp, s)
        plsc.subcore_barrier()
        scatter_fn()
        plsc.subcore_barrier()
        pltpu.sync_copy(acc_sp.at[rows], out_ref.at[rows])
        plsc.subcore_barrier()

    def ct_pass(sct, dct, ct_out):
        pltpu.sync_copy(dct.at[s], cidx1)
        pltpu.sync_copy(sct.at[s], crow2)
        acc_pass(lambda: pipe_scatter(oht, cidx1, crow2, NCH_W), ct_out)

    @pl.when(c == 0)
    def _():
        acc_pass(lambda: pipe_scatter(xs0, gidx1, srow2, NCH_C), y0_out)
        acc_pass(lambda: pipe_scatter(xs1, gidx1, srow2, NCH_C), y1_out)
        ct_pass(sct0, dct0, ct0_out)

    @pl.when(c == 1)
    def _():
        acc_pass(lambda: pipe_scatter(xs2, gidx1, srow2, NCH_C), y2_out)
        acc_pass(lambda: pipe_scatter(xs3, gidx1, srow2, NCH_C), y3_out)
        ct_pass(sct1, dct1, ct1_out)


_BLK = 1000
_NBLK = N // _BLK


def _prep_body(ind0_ref, ind1_ref, x_ref, batch_ref, xs0_ref, xs1_ref,
               xs2_ref, xs3_ref, dinv_ref, cntinv_ref, oht_ref, cnt_acc):
    i = pl.program_id(0)
    deg = ind0_ref[:, 0:1] + ind1_ref[:, 0:1] + 1.0
    dinv = lax.rsqrt(deg)
    dinv_ref[...] = dinv
    xs = x_ref[...] * dinv
    xs0_ref[...] = xs[:, 0:64]
    xs1_ref[...] = xs[:, 64:128]
    xs2_ref[...] = xs[:, 128:192]
    xs3_ref[...] = xs[:, 192:256]
    onehot = (batch_ref[...] ==
              lax.broadcasted_iota(jnp.int32, (_BLK, G), 1)).astype(jnp.float32)
    oht_ref[...] = onehot * dinv

    @pl.when(i == 0)
    def _():
        cnt_acc[...] = jnp.zeros_like(cnt_acc)

    cnt_acc[...] += jnp.sum(onehot, axis=0, keepdims=True)

    @pl.when(i == _NBLK - 1)
    def _():
        cntinv_ref[...] = 1.0 / jnp.maximum(cnt_acc[...], 1.0)


_prep_call = pl.pallas_call(
    _prep_body,
    grid=(_NBLK,),
    in_specs=[
        pl.BlockSpec((_BLK, 16), lambda i: (i, 0)),
        pl.BlockSpec((_BLK, 16), lambda i: (i, 0)),
        pl.BlockSpec((_BLK, DIN), lambda i: (i, 0)),
        pl.BlockSpec((_BLK, 1), lambda i: (i, 0)),
    ],
    out_specs=[
        pl.BlockSpec((_BLK, G), lambda i: (i, 0)),
        pl.BlockSpec((_BLK, G), lambda i: (i, 0)),
        pl.BlockSpec((_BLK, G), lambda i: (i, 0)),
        pl.BlockSpec((_BLK, G), lambda i: (i, 0)),
        pl.BlockSpec((_BLK, 1), lambda i: (i, 0)),
        pl.BlockSpec((1, G), lambda i: (0, 0)),
        pl.BlockSpec((_BLK, G), lambda i: (i, 0)),
    ],
    out_shape=[
        jax.ShapeDtypeStruct((N, G), jnp.float32),
        jax.ShapeDtypeStruct((N, G), jnp.float32),
        jax.ShapeDtypeStruct((N, G), jnp.float32),
        jax.ShapeDtypeStruct((N, G), jnp.float32),
        jax.ShapeDtypeStruct((N, 1), jnp.float32),
        jax.ShapeDtypeStruct((1, G), jnp.float32),
        jax.ShapeDtypeStruct((N, G), jnp.float32),
    ],
    scratch_shapes=[pltpu.VMEM((1, G), jnp.float32)],
    compiler_params=pltpu.CompilerParams(
        dimension_semantics=("arbitrary",)),
)


def _dense_body(x_ref, y0_ref, y1_ref, y2_ref, y3_ref, ct0_ref, ct1_ref,
                dinv_ref, batch_ref,
                cntinv_ref, w1_ref, b1_ref, w3_ref, b3_ref,
                w2_ref, b2_ref, w4_ref, b4_ref,
                zm_ref, zl_ref, accm, accl):
    i = pl.program_id(0)
    dinv = dinv_ref[...]
    dinv2 = dinv * dinv
    y_acc = jnp.concatenate(
        [y0_ref[...], y1_ref[...], y2_ref[...], y3_ref[...]], axis=1)
    y = dinv * y_acc + dinv2 * x_ref[...]
    hm = jnp.maximum(
        jnp.dot(y, w1_ref[...], preferred_element_type=jnp.float32)
        + b1_ref[...], 0.0)
    hl = jnp.maximum(
        jnp.dot(y, w3_ref[...], preferred_element_type=jnp.float32)
        + b3_ref[...], 0.0)
    onehot = (batch_ref[...] ==
              lax.broadcasted_iota(jnp.int32, (_BLK, G), 1)).astype(jnp.float32)
    ctf = dinv * (ct0_ref[...] + ct1_ref[...]) + dinv2 * onehot
    dn = (((0,), (0,)), ((), ()))

    @pl.when(i == 0)
    def _():
        accm[...] = jnp.zeros_like(accm)
        accl[...] = jnp.zeros_like(accl)

    accm[...] += lax.dot_general(ctf, hm, dimension_numbers=dn,
                                 preferred_element_type=jnp.float32)
    accl[...] += lax.dot_general(ctf, hl, dimension_numbers=dn,
                                 preferred_element_type=jnp.float32)

    @pl.when(i == _NBLK - 1)
    def _():
        cntinv = cntinv_ref[...]
        zm_ref[...] = jnp.dot(accm[...] * cntinv, w2_ref[...],
                              preferred_element_type=jnp.float32) + b2_ref[...]
        zl_ref[...] = jnp.dot(accl[...] * cntinv, w4_ref[...],
                              preferred_element_type=jnp.float32) + b4_ref[...]


_dense_call = pl.pallas_call(
    _dense_body,
    grid=(_NBLK,),
    in_specs=[
        pl.BlockSpec((_BLK, DIN), lambda i: (i, 0)),
        pl.BlockSpec((_BLK, G), lambda i: (i, 0)),
        pl.BlockSpec((_BLK, G), lambda i: (i, 0)),
        pl.BlockSpec((_BLK, G), lambda i: (i, 0)),
        pl.BlockSpec((_BLK, G), lambda i: (i, 0)),
        pl.BlockSpec((_BLK, G), lambda i: (i, 0)),
        pl.BlockSpec((_BLK, G), lambda i: (i, 0)),
        pl.BlockSpec((_BLK, 1), lambda i: (i, 0)),
        pl.BlockSpec((_BLK, 1), lambda i: (i, 0)),
        pl.BlockSpec((G, 1), lambda i: (0, 0)),
        pl.BlockSpec((DIN, DH), lambda i: (0, 0)),
        pl.BlockSpec((1, DH), lambda i: (0, 0)),
        pl.BlockSpec((DIN, DH), lambda i: (0, 0)),
        pl.BlockSpec((1, DH), lambda i: (0, 0)),
        pl.BlockSpec((DH, DZ), lambda i: (0, 0)),
        pl.BlockSpec((1, DZ), lambda i: (0, 0)),
        pl.BlockSpec((DH, DZ), lambda i: (0, 0)),
        pl.BlockSpec((1, DZ), lambda i: (0, 0)),
    ],
    out_specs=[
        pl.BlockSpec((G, DZ), lambda i: (0, 0)),
        pl.BlockSpec((G, DZ), lambda i: (0, 0)),
    ],
    out_shape=[
        jax.ShapeDtypeStruct((G, DZ), jnp.float32),
        jax.ShapeDtypeStruct((G, DZ), jnp.float32),
    ],
    scratch_shapes=[
        pltpu.VMEM((G, DH), jnp.float32),
        pltpu.VMEM((G, DH), jnp.float32),
    ],
    compiler_params=pltpu.CompilerParams(
        dimension_semantics=("arbitrary",)),
)


def _pad_ids(ids, per_worker, nworkers, pad_base, nch):
    padded = nch * CH
    npad = padded - per_worker
    padv = pad_base + (jnp.arange(npad, dtype=jnp.int32) % 16)
    padv = jnp.broadcast_to(padv, (nworkers, npad))
    return jnp.concatenate([ids.reshape(nworkers, per_worker), padv], axis=1)


def kernel(x, edge_index, batch, W1, b1, W2, b2, W3, b3, W4, b4):
    src = edge_index[0].astype(jnp.int32)
    dst = edge_index[1].astype(jnp.int32)
    batch = batch.astype(jnp.int32)

    dst_deg = _pad_ids(dst, EW, NW, N, NCH_W).reshape(NW, NCH_W, CH)
    srcy = _pad_ids(src, EC, 16, 0, NCH_C)
    dsty = _pad_ids(dst, EC, 16, N, NCH_C).reshape(16, NCH_C, CH)
    sct = _pad_ids(src, EW, NW, N, NCH_W).reshape(2, 16, NCH_W, CH)
    dct = _pad_ids(dst, EW, NW, 0, NCH_W).reshape(2, 16, NCH_W * CH)

    zeros64 = jnp.zeros((128, G), jnp.float32)
    zeros16 = jnp.zeros((128, 16), jnp.float32)
    onehot16 = jnp.zeros((CH, 16), jnp.float32).at[:, 0].set(1.0)

    deg_raw = _deg_kernel(dst_deg, onehot16, zeros16)

    xs0, xs1, xs2, xs3, dinv, cntinv, oht = _prep_call(
        deg_raw[:NPAD], deg_raw[NPAD:], x, batch.reshape(N, 1))

    y0p, y1p, y2p, y3p, ct0p, ct1p = _scatter_kernel(
        xs0, xs1, xs2, xs3, oht, srcy, dsty, sct[0], sct[1],
        dct[0], dct[1], zeros64)

    z_mean, z_logvar = _dense_call(
        x, y0p[:N], y1p[:N], y2p[:N], y3p[:N], ct0p[:N], ct1p[:N], dinv,
        batch.reshape(N, 1),
        cntinv.reshape(G, 1),
        W1, b1.reshape(1, DH), W3, b3.reshape(1, DH),
        W2, b2.reshape(1, DZ), W4, b4.reshape(1, DZ))
    return (z_mean, z_logvar)


# speedup vs baseline: 29.1232x; 1.0365x over previous
import functools

import jax
import jax.numpy as jnp
from jax import lax
from jax.experimental import pallas as pl
from jax.experimental.pallas import tpu as pltpu
from jax.experimental.pallas import tpu_sc as plsc

N = 10000
E = 160000
DIN = 256
DH = 512
DZ = 128
G = 64
NPAD = 10240
NW = 32
EW = E // NW
EC = E // 16
CH = 128
NCH_W = 40
NCH_C = 80
NBUF = 4
ROWS_PER_SUB = NPAD // 16

_mesh = plsc.VectorSubcoreMesh(core_axis_name="c", subcore_axis_name="s")


def _zero_rows(zeros_hbm, sp_ref, s):
    base = s * ROWS_PER_SUB
    for k in range(ROWS_PER_SUB // 128):
        pltpu.sync_copy(zeros_hbm, sp_ref.at[pl.ds(base + k * 128, 128)])


@functools.partial(
    pl.kernel,
    out_type=jax.ShapeDtypeStruct((2 * NPAD, 16), jnp.float32),
    mesh=_mesh,
    scratch_types=[
        pltpu.VMEM((NCH_W, CH), jnp.int32),
        pltpu.VMEM((CH, 16), jnp.float32),
        pltpu.VMEM_SHARED((NPAD, 16), jnp.float32),
        pltpu.SemaphoreType.DMA,
    ],
    compiler_params=pltpu.CompilerParams(
        needs_layout_passes=False, use_tc_tiling_on_sc=False),
)
def _deg_kernel(dst_hbm, onehot_hbm, zeros16_hbm, out_hbm, d2, oh, deg_sp,
                sem):
    c = lax.axis_index("c")
    s = lax.axis_index("s")
    _zero_rows(zeros16_hbm, deg_sp, s)
    pltpu.sync_copy(onehot_hbm, oh)
    pltpu.sync_copy(dst_hbm.at[c * 16 + s], d2)
    plsc.subcore_barrier()

    def body(t, carry):
        for b in range(8):
            pltpu.async_copy(oh, deg_sp.at[d2.at[t * 8 + b]], sem, add=True)
        for b in range(8):
            pltpu.make_async_copy(onehot_hbm, oh, sem).wait()
        return carry

    lax.fori_loop(0, NCH_W // 8, body, 0)
    plsc.subcore_barrier()
    pltpu.sync_copy(deg_sp.at[pl.ds(s * ROWS_PER_SUB, ROWS_PER_SUB)],
                    out_hbm.at[pl.ds(c * NPAD + s * ROWS_PER_SUB,
                                     ROWS_PER_SUB)])


_QSHAPE = jax.ShapeDtypeStruct((NPAD, G), jnp.float32)


@functools.partial(
    pl.kernel,
    out_type=(
        _QSHAPE, _QSHAPE, _QSHAPE, _QSHAPE,
        _QSHAPE, _QSHAPE,
    ),
    mesh=_mesh,
    scratch_types=[
        pltpu.VMEM((CH * NCH_C,), jnp.int32),
        pltpu.VMEM((NCH_C, CH), jnp.int32),
        pltpu.VMEM((CH * NCH_W,), jnp.int32),
        pltpu.VMEM((NCH_W, CH), jnp.int32),
        [pltpu.VMEM((CH, G), jnp.float32) for _ in range(NBUF)],
        pltpu.VMEM((N,), jnp.int32),
        pltpu.VMEM((N,), jnp.float32),
        pltpu.VMEM_SHARED((NPAD, G), jnp.float32),
        pltpu.SemaphoreType.DMA,
        pltpu.SemaphoreType.DMA,
    ],
    compiler_params=pltpu.CompilerParams(
        needs_layout_passes=False, use_tc_tiling_on_sc=False),
)
def _scatter_kernel(xs0, xs1, xs2, xs3, srcy, dsty, sct0, sct1,
                    dct0, dct1, batch_hbm, dinv_hbm, zeros_hbm,
                    y0_out, y1_out, y2_out, y3_out, ct0_out, ct1_out,
                    gidx1, srow2, cidx1, crow2, ring, btab, dtab,
                    acc_sp, gsem, ssem):
    ohb = ring[:2]
    c = lax.axis_index("c")
    s = lax.axis_index("s")
    iota = lax.iota(jnp.int32, 16)
    rows = pl.ds(s * ROWS_PER_SUB, ROWS_PER_SUB)

    pltpu.sync_copy(srcy.at[s], gidx1)
    pltpu.sync_copy(dsty.at[s], srow2)
    pltpu.sync_copy(batch_hbm, btab)
    pltpu.sync_copy(dinv_hbm, dtab)

    def pipe_scatter(tab_ref, idx1, row2, nch):
        half = NBUF // 2
        nstep = nch // half

        def step(t, carry):
            def run_set(base):
                bufs = [ring[base + b] for b in range(half)]

                @pl.when(t >= 2)
                def _():
                    for b in range(half):
                        pltpu.make_async_copy(zeros_hbm, bufs[b], ssem).wait()

                hs = []
                for b in range(half):
                    j = t * half + b
                    hs.append(pltpu.async_copy(
                        tab_ref.at[idx1.at[pl.ds(j * CH, CH)]],
                        bufs[b], gsem))
                for h in hs:
                    h.wait()
                for b in range(half):
                    j = t * half + b
                    pltpu.async_copy(bufs[b], acc_sp.at[row2.at[j]],
                                     ssem, add=True)

            @pl.when(t % 2 == 0)
            def _():
                run_set(0)

            @pl.when(t % 2 == 1)
            def _():
                run_set(half)

            return carry

        lax.fori_loop(0, nstep, step, 0)
        for b in range(NBUF):
            pltpu.make_async_copy(zeros_hbm, ring[b], ssem).wait()

    def acc_pass(scatter_fn, out_ref):
        _zero_rows(zeros_hbm, acc_sp, s)
        plsc.subcore_barrier()
        scatter_fn()
        plsc.subcore_barrier()
        pltpu.sync_copy(acc_sp.at[rows], out_ref.at[rows])
        plsc.subcore_barrier()

    def ct_build(k, p, clear):
        for v in range(8):
            d = cidx1[pl.ds(k * CH + v * 16, 16)]
            gi = plsc.load_gather(btab, [d])
            e = v * 16 + iota
            if clear:
                plsc.store_scatter(ohb[p], [e, gi],
                                   jnp.zeros((16,), jnp.float32))
            else:
                val = plsc.load_gather(dtab, [d])
                plsc.store_scatter(ohb[p], [e, gi], val)

    def ct_scalar():
        def body(k, carry):
            def run(p):
                @pl.when(k >= 2)
                def _():
                    pltpu.make_async_copy(zeros_hbm, ohb[p], ssem).wait()
                    ct_build(k - 2, p, True)

                ct_build(k, p, False)
                pltpu.async_copy(ohb[p], acc_sp.at[crow2.at[k]],
                                 ssem, add=True)

            @pl.when(k % 2 == 0)
            def _():
                run(0)

            @pl.when(k % 2 == 1)
            def _():
                run(1)

            return carry

        lax.fori_loop(0, NCH_W, body, 0)
        for p in range(2):
            pltpu.make_async_copy(zeros_hbm, ohb[p], ssem).wait()

    def ct_pass(sct, dct, ct_out):
        pltpu.sync_copy(dct.at[s], cidx1)
        pltpu.sync_copy(sct.at[s], crow2)
        pltpu.sync_copy(zeros_hbm, ohb[0])
        pltpu.sync_copy(zeros_hbm, ohb[1])
        acc_pass(ct_scalar, ct_out)

    @pl.when(c == 0)
    def _():
        acc_pass(lambda: pipe_scatter(xs0, gidx1, srow2, NCH_C), y0_out)
        acc_pass(lambda: pipe_scatter(xs1, gidx1, srow2, NCH_C), y1_out)
        ct_pass(sct0, dct0, ct0_out)

    @pl.when(c == 1)
    def _():
        acc_pass(lambda: pipe_scatter(xs2, gidx1, srow2, NCH_C), y2_out)
        acc_pass(lambda: pipe_scatter(xs3, gidx1, srow2, NCH_C), y3_out)
        ct_pass(sct1, dct1, ct1_out)


_BLK = 1000
_NBLK = N // _BLK


def _prep_body(ind0_ref, ind1_ref, x_ref, batch_ref, xs0_ref, xs1_ref,
               xs2_ref, xs3_ref, dinv_ref, cntinv_ref, cnt_acc):
    i = pl.program_id(0)
    deg = ind0_ref[:, 0:1] + ind1_ref[:, 0:1] + 1.0
    dinv = lax.rsqrt(deg)
    dinv_ref[...] = dinv
    xs = x_ref[...] * dinv
    xs0_ref[...] = xs[:, 0:64]
    xs1_ref[...] = xs[:, 64:128]
    xs2_ref[...] = xs[:, 128:192]
    xs3_ref[...] = xs[:, 192:256]
    onehot = (batch_ref[...] ==
              lax.broadcasted_iota(jnp.int32, (_BLK, G), 1)).astype(jnp.float32)

    @pl.when(i == 0)
    def _():
        cnt_acc[...] = jnp.zeros_like(cnt_acc)

    cnt_acc[...] += jnp.sum(onehot, axis=0, keepdims=True)

    @pl.when(i == _NBLK - 1)
    def _():
        cntinv_ref[...] = 1.0 / jnp.maximum(cnt_acc[...], 1.0)


_prep_call = pl.pallas_call(
    _prep_body,
    grid=(_NBLK,),
    in_specs=[
        pl.BlockSpec((_BLK, 16), lambda i: (i, 0)),
        pl.BlockSpec((_BLK, 16), lambda i: (i, 0)),
        pl.BlockSpec((_BLK, DIN), lambda i: (i, 0)),
        pl.BlockSpec((_BLK, 1), lambda i: (i, 0)),
    ],
    out_specs=[
        pl.BlockSpec((_BLK, G), lambda i: (i, 0)),
        pl.BlockSpec((_BLK, G), lambda i: (i, 0)),
        pl.BlockSpec((_BLK, G), lambda i: (i, 0)),
        pl.BlockSpec((_BLK, G), lambda i: (i, 0)),
        pl.BlockSpec((_BLK, 1), lambda i: (i, 0)),
        pl.BlockSpec((1, G), lambda i: (0, 0)),
    ],
    out_shape=[
        jax.ShapeDtypeStruct((N, G), jnp.float32),
        jax.ShapeDtypeStruct((N, G), jnp.float32),
        jax.ShapeDtypeStruct((N, G), jnp.float32),
        jax.ShapeDtypeStruct((N, G), jnp.float32),
        jax.ShapeDtypeStruct((N, 1), jnp.float32),
        jax.ShapeDtypeStruct((1, G), jnp.float32),
    ],
    scratch_shapes=[pltpu.VMEM((1, G), jnp.float32)],
    compiler_params=pltpu.CompilerParams(
        dimension_semantics=("arbitrary",)),
)


def _dense_body(x_ref, y0_ref, y1_ref, y2_ref, y3_ref, ct0_ref, ct1_ref,
                dinv_ref, batch_ref,
                cntinv_ref, w1_ref, b1_ref, w3_ref, b3_ref,
                w2_ref, b2_ref, w4_ref, b4_ref,
                zm_ref, zl_ref, accm, accl):
    i = pl.program_id(0)
    dinv = dinv_ref[...]
    dinv2 = dinv * dinv
    y_acc = jnp.concatenate(
        [y0_ref[...], y1_ref[...], y2_ref[...], y3_ref[...]], axis=1)
    y = dinv * y_acc + dinv2 * x_ref[...]
    hm = jnp.maximum(
        jnp.dot(y, w1_ref[...], preferred_element_type=jnp.float32)
        + b1_ref[...], 0.0)
    hl = jnp.maximum(
        jnp.dot(y, w3_ref[...], preferred_element_type=jnp.float32)
        + b3_ref[...], 0.0)
    onehot = (batch_ref[...] ==
              lax.broadcasted_iota(jnp.int32, (_BLK, G), 1)).astype(jnp.float32)
    ctf = dinv * (ct0_ref[...] + ct1_ref[...]) + dinv2 * onehot
    dn = (((0,), (0,)), ((), ()))

    @pl.when(i == 0)
    def _():
        accm[...] = jnp.zeros_like(accm)
        accl[...] = jnp.zeros_like(accl)

    accm[...] += lax.dot_general(ctf, hm, dimension_numbers=dn,
                                 preferred_element_type=jnp.float32)
    accl[...] += lax.dot_general(ctf, hl, dimension_numbers=dn,
                                 preferred_element_type=jnp.float32)

    @pl.when(i == _NBLK - 1)
    def _():
        cntinv = cntinv_ref[...]
        zm_ref[...] = jnp.dot(accm[...] * cntinv, w2_ref[...],
                              preferred_element_type=jnp.float32) + b2_ref[...]
        zl_ref[...] = jnp.dot(accl[...] * cntinv, w4_ref[...],
                              preferred_element_type=jnp.float32) + b4_ref[...]


_dense_call = pl.pallas_call(
    _dense_body,
    grid=(_NBLK,),
    in_specs=[
        pl.BlockSpec((_BLK, DIN), lambda i: (i, 0)),
        pl.BlockSpec((_BLK, G), lambda i: (i, 0)),
        pl.BlockSpec((_BLK, G), lambda i: (i, 0)),
        pl.BlockSpec((_BLK, G), lambda i: (i, 0)),
        pl.BlockSpec((_BLK, G), lambda i: (i, 0)),
        pl.BlockSpec((_BLK, G), lambda i: (i, 0)),
        pl.BlockSpec((_BLK, G), lambda i: (i, 0)),
        pl.BlockSpec((_BLK, 1), lambda i: (i, 0)),
        pl.BlockSpec((_BLK, 1), lambda i: (i, 0)),
        pl.BlockSpec((G, 1), lambda i: (0, 0)),
        pl.BlockSpec((DIN, DH), lambda i: (0, 0)),
        pl.BlockSpec((1, DH), lambda i: (0, 0)),
        pl.BlockSpec((DIN, DH), lambda i: (0, 0)),
        pl.BlockSpec((1, DH), lambda i: (0, 0)),
        pl.BlockSpec((DH, DZ), lambda i: (0, 0)),
        pl.BlockSpec((1, DZ), lambda i: (0, 0)),
        pl.BlockSpec((DH, DZ), lambda i: (0, 0)),
        pl.BlockSpec((1, DZ), lambda i: (0, 0)),
    ],
    out_specs=[
        pl.BlockSpec((G, DZ), lambda i: (0, 0)),
        pl.BlockSpec((G, DZ), lambda i: (0, 0)),
    ],
    out_shape=[
        jax.ShapeDtypeStruct((G, DZ), jnp.float32),
        jax.ShapeDtypeStruct((G, DZ), jnp.float32),
    ],
    scratch_shapes=[
        pltpu.VMEM((G, DH), jnp.float32),
        pltpu.VMEM((G, DH), jnp.float32),
    ],
    compiler_params=pltpu.CompilerParams(
        dimension_semantics=("arbitrary",)),
)


def _pad_ids(ids, per_worker, nworkers, pad_base, nch):
    padded = nch * CH
    npad = padded - per_worker
    padv = pad_base + (jnp.arange(npad, dtype=jnp.int32) % 16)
    padv = jnp.broadcast_to(padv, (nworkers, npad))
    return jnp.concatenate([ids.reshape(nworkers, per_worker), padv], axis=1)


def kernel(x, edge_index, batch, W1, b1, W2, b2, W3, b3, W4, b4):
    src = edge_index[0].astype(jnp.int32)
    dst = edge_index[1].astype(jnp.int32)
    batch = batch.astype(jnp.int32)

    dst_deg = _pad_ids(dst, EW, NW, N, NCH_W).reshape(NW, NCH_W, CH)
    srcy = _pad_ids(src, EC, 16, 0, NCH_C)
    dsty = _pad_ids(dst, EC, 16, N, NCH_C).reshape(16, NCH_C, CH)
    sct = _pad_ids(src, EW, NW, N, NCH_W).reshape(2, 16, NCH_W, CH)
    dct = _pad_ids(dst, EW, NW, 0, NCH_W).reshape(2, 16, NCH_W * CH)

    zeros64 = jnp.zeros((128, G), jnp.float32)
    zeros16 = jnp.zeros((128, 16), jnp.float32)
    onehot16 = jnp.zeros((CH, 16), jnp.float32).at[:, 0].set(1.0)

    deg_raw = _deg_kernel(dst_deg, onehot16, zeros16)

    xs0, xs1, xs2, xs3, dinv, cntinv = _prep_call(
        deg_raw[:NPAD], deg_raw[NPAD:], x, batch.reshape(N, 1))

    y0p, y1p, y2p, y3p, ct0p, ct1p = _scatter_kernel(
        xs0, xs1, xs2, xs3, srcy, dsty, sct[0], sct[1],
        dct[0], dct[1], batch, dinv.reshape(N), zeros64)

    z_mean, z_logvar = _dense_call(
        x, y0p[:N], y1p[:N], y2p[:N], y3p[:N], ct0p[:N], ct1p[:N], dinv,
        batch.reshape(N, 1),
        cntinv.reshape(G, 1),
        W1, b1.reshape(1, DH), W3, b3.reshape(1, DH),
        W2, b2.reshape(1, DZ), W4, b4.reshape(1, DZ))
    return (z_mean, z_logvar)


# speedup vs baseline: 34.7931x; 1.1947x over previous
import functools

import jax
import jax.numpy as jnp
from jax import lax
from jax.experimental import pallas as pl
from jax.experimental.pallas import tpu as pltpu
from jax.experimental.pallas import tpu_sc as plsc

N = 10000
E = 160000
DIN = 256
DH = 512
DZ = 128
G = 64
NPAD = 10240
NW = 32
EW = E // NW
EC = E // 16
CH = 128
NCH_W = 40
NCH_C = 80
NBUF = 4
ROWS_PER_SUB = NPAD // 16

_mesh = plsc.VectorSubcoreMesh(core_axis_name="c", subcore_axis_name="s")


def _zero_rows(zeros_hbm, sp_ref, s):
    pltpu.sync_copy(zeros_hbm, sp_ref.at[pl.ds(s * ROWS_PER_SUB,
                                               ROWS_PER_SUB)])


@functools.partial(
    pl.kernel,
    out_type=jax.ShapeDtypeStruct((2 * NPAD, 16), jnp.float32),
    mesh=_mesh,
    scratch_types=[
        pltpu.VMEM((NCH_W, CH), jnp.int32),
        pltpu.VMEM((CH, 16), jnp.float32),
        pltpu.VMEM_SHARED((NPAD, 16), jnp.float32),
        pltpu.SemaphoreType.DMA,
    ],
    compiler_params=pltpu.CompilerParams(
        needs_layout_passes=False, use_tc_tiling_on_sc=False),
)
def _deg_kernel(dst_hbm, onehot_hbm, zeros16_hbm, out_hbm, d2, oh, deg_sp,
                sem):
    c = lax.axis_index("c")
    s = lax.axis_index("s")
    _zero_rows(zeros16_hbm, deg_sp, s)
    pltpu.sync_copy(onehot_hbm, oh)
    pltpu.sync_copy(dst_hbm.at[c * 16 + s], d2)
    plsc.subcore_barrier()

    def body(t, carry):
        for b in range(8):
            pltpu.async_copy(oh, deg_sp.at[d2.at[t * 8 + b]], sem, add=True)
        for b in range(8):
            pltpu.make_async_copy(onehot_hbm, oh, sem).wait()
        return carry

    lax.fori_loop(0, NCH_W // 8, body, 0)
    plsc.subcore_barrier()
    pltpu.sync_copy(deg_sp.at[pl.ds(s * ROWS_PER_SUB, ROWS_PER_SUB)],
                    out_hbm.at[pl.ds(c * NPAD + s * ROWS_PER_SUB,
                                     ROWS_PER_SUB)])


_HSHAPE = jax.ShapeDtypeStruct((NPAD, 128), jnp.float32)


@functools.partial(
    pl.kernel,
    out_type=(
        _HSHAPE,
        _HSHAPE,
        _HSHAPE,
    ),
    mesh=_mesh,
    scratch_types=[
        pltpu.VMEM((CH * NCH_C,), jnp.int32),
        pltpu.VMEM((NCH_C, CH), jnp.int32),
        pltpu.VMEM((CH * NCH_W,), jnp.int32),
        pltpu.VMEM((NCH_W, CH), jnp.int32),
        [pltpu.VMEM((CH, G), jnp.float32) for _ in range(NBUF)],
        pltpu.VMEM((N,), jnp.int32),
        pltpu.VMEM((N,), jnp.float32),
        pltpu.VMEM_SHARED((NPAD, G), jnp.float32),
        pltpu.SemaphoreType.DMA,
        pltpu.SemaphoreType.DMA,
    ],
    compiler_params=pltpu.CompilerParams(
        needs_layout_passes=False, use_tc_tiling_on_sc=False),
)
def _scatter_kernel(xs0, xs1, xs2, xs3, srcy, dsty, sct0, sct1,
                    dct0, dct1, batch_hbm, dinv_hbm, zeros_hbm,
                    y01_out, y23_out, ct01_out,
                    gidx1, srow2, cidx1, crow2, ring, btab, dtab,
                    acc_sp, gsem, ssem):
    ohb = ring[:2]
    c = lax.axis_index("c")
    s = lax.axis_index("s")
    iota = lax.iota(jnp.int32, 16)
    rows = pl.ds(s * ROWS_PER_SUB, ROWS_PER_SUB)

    pltpu.sync_copy(srcy.at[s], gidx1)
    pltpu.sync_copy(dsty.at[s], srow2)
    pltpu.sync_copy(batch_hbm, btab)
    pltpu.sync_copy(dinv_hbm, dtab)

    def pipe_scatter(tab_ref, idx1, row2, nch):
        half = NBUF // 2
        nstep = nch // half

        def step(t, carry):
            def run_set(base):
                bufs = [ring[base + b] for b in range(half)]

                @pl.when(t >= 2)
                def _():
                    for b in range(half):
                        pltpu.make_async_copy(zeros_hbm.at[pl.ds(0, CH)],
                                              bufs[b], ssem).wait()

                hs = []
                for b in range(half):
                    j = t * half + b
                    hs.append(pltpu.async_copy(
                        tab_ref.at[idx1.at[pl.ds(j * CH, CH)]],
                        bufs[b], gsem))
                for h in hs:
                    h.wait()
                for b in range(half):
                    j = t * half + b
                    pltpu.async_copy(bufs[b], acc_sp.at[row2.at[j]],
                                     ssem, add=True)

            @pl.when(t % 2 == 0)
            def _():
                run_set(0)

            @pl.when(t % 2 == 1)
            def _():
                run_set(half)

            return carry

        lax.fori_loop(0, nstep, step, 0)
        for b in range(NBUF):
            pltpu.make_async_copy(zeros_hbm.at[pl.ds(0, CH)],
                                  ring[b], ssem).wait()

    def acc_pass(scatter_fn, out_ref, col0):
        _zero_rows(zeros_hbm, acc_sp, s)
        plsc.subcore_barrier()
        scatter_fn()
        plsc.subcore_barrier()
        pltpu.sync_copy(acc_sp.at[rows], out_ref.at[rows, pl.ds(col0, G)])
        plsc.subcore_barrier()

    def ct_build(k, p, clear):
        for v in range(8):
            d = cidx1[pl.ds(k * CH + v * 16, 16)]
            gi = plsc.load_gather(btab, [d])
            e = v * 16 + iota
            if clear:
                plsc.store_scatter(ohb[p], [e, gi],
                                   jnp.zeros((16,), jnp.float32))
            else:
                val = plsc.load_gather(dtab, [d])
                plsc.store_scatter(ohb[p], [e, gi], val)

    def ct_scalar():
        def body(k, carry):
            def run(p):
                @pl.when(k >= 2)
                def _():
                    pltpu.make_async_copy(zeros_hbm.at[pl.ds(0, CH)],
                                          ohb[p], ssem).wait()
                    ct_build(k - 2, p, True)

                ct_build(k, p, False)
                pltpu.async_copy(ohb[p], acc_sp.at[crow2.at[k]],
                                 ssem, add=True)

            @pl.when(k % 2 == 0)
            def _():
                run(0)

            @pl.when(k % 2 == 1)
            def _():
                run(1)

            return carry

        lax.fori_loop(0, NCH_W, body, 0)
        for p in range(2):
            pltpu.make_async_copy(zeros_hbm.at[pl.ds(0, CH)],
                                  ohb[p], ssem).wait()

    def ct_pass(sct, dct, col0):
        pltpu.sync_copy(dct.at[s], cidx1)
        pltpu.sync_copy(sct.at[s], crow2)
        pltpu.sync_copy(zeros_hbm.at[pl.ds(0, CH)], ohb[0])
        pltpu.sync_copy(zeros_hbm.at[pl.ds(0, CH)], ohb[1])
        acc_pass(ct_scalar, ct01_out, col0)

    @pl.when(c == 0)
    def _():
        acc_pass(lambda: pipe_scatter(xs0, gidx1, srow2, NCH_C), y01_out, 0)
        acc_pass(lambda: pipe_scatter(xs1, gidx1, srow2, NCH_C), y01_out, G)
        ct_pass(sct0, dct0, 0)

    @pl.when(c == 1)
    def _():
        acc_pass(lambda: pipe_scatter(xs2, gidx1, srow2, NCH_C), y23_out, 0)
        acc_pass(lambda: pipe_scatter(xs3, gidx1, srow2, NCH_C), y23_out, G)
        ct_pass(sct1, dct1, G)


_BLK = 2000
_NBLK = N // _BLK


def _prep_body(ind0_ref, ind1_ref, x_ref, batch_ref, xs0_ref, xs1_ref,
               xs2_ref, xs3_ref, dinv_ref, cntinv_ref, cnt_acc):
    i = pl.program_id(0)
    deg = ind0_ref[:, 0:1] + ind1_ref[:, 0:1] + 1.0
    dinv = lax.rsqrt(deg)
    dinv_ref[...] = dinv
    xs = x_ref[...] * dinv
    xs0_ref[...] = xs[:, 0:64]
    xs1_ref[...] = xs[:, 64:128]
    xs2_ref[...] = xs[:, 128:192]
    xs3_ref[...] = xs[:, 192:256]
    onehot = (batch_ref[...] ==
              lax.broadcasted_iota(jnp.int32, (_BLK, G), 1)).astype(jnp.float32)

    @pl.when(i == 0)
    def _():
        cnt_acc[...] = jnp.zeros_like(cnt_acc)

    cnt_acc[...] += jnp.sum(onehot, axis=0, keepdims=True)

    @pl.when(i == _NBLK - 1)
    def _():
        cntinv_ref[...] = 1.0 / jnp.maximum(cnt_acc[...], 1.0)


_prep_call = pl.pallas_call(
    _prep_body,
    grid=(_NBLK,),
    in_specs=[
        pl.BlockSpec((_BLK, 16), lambda i: (i, 0)),
        pl.BlockSpec((_BLK, 16), lambda i: (i, 0)),
        pl.BlockSpec((_BLK, DIN), lambda i: (i, 0)),
        pl.BlockSpec((_BLK, 1), lambda i: (i, 0)),
    ],
    out_specs=[
        pl.BlockSpec((_BLK, G), lambda i: (i, 0)),
        pl.BlockSpec((_BLK, G), lambda i: (i, 0)),
        pl.BlockSpec((_BLK, G), lambda i: (i, 0)),
        pl.BlockSpec((_BLK, G), lambda i: (i, 0)),
        pl.BlockSpec((_BLK, 1), lambda i: (i, 0)),
        pl.BlockSpec((1, G), lambda i: (0, 0)),
    ],
    out_shape=[
        jax.ShapeDtypeStruct((N, G), jnp.float32),
        jax.ShapeDtypeStruct((N, G), jnp.float32),
        jax.ShapeDtypeStruct((N, G), jnp.float32),
        jax.ShapeDtypeStruct((N, G), jnp.float32),
        jax.ShapeDtypeStruct((N, 1), jnp.float32),
        jax.ShapeDtypeStruct((1, G), jnp.float32),
    ],
    scratch_shapes=[pltpu.VMEM((1, G), jnp.float32)],
    compiler_params=pltpu.CompilerParams(
        dimension_semantics=("arbitrary",)),
)


def _dense_body(x_ref, y01_ref, y23_ref, ct01_ref,
                dinv_ref, batch_ref,
                cntinv_ref, w1_ref, b1_ref, w3_ref, b3_ref,
                w2_ref, b2_ref, w4_ref, b4_ref,
                zm_ref, zl_ref, accm, accl):
    i = pl.program_id(0)
    dinv = dinv_ref[...]
    dinv2 = dinv * dinv
    y_acc = jnp.concatenate([y01_ref[...], y23_ref[...]], axis=1)
    y = dinv * y_acc + dinv2 * x_ref[...]
    hm = jnp.maximum(
        jnp.dot(y, w1_ref[...], preferred_element_type=jnp.float32)
        + b1_ref[...], 0.0)
    hl = jnp.maximum(
        jnp.dot(y, w3_ref[...], preferred_element_type=jnp.float32)
        + b3_ref[...], 0.0)
    onehot = (batch_ref[...] ==
              lax.broadcasted_iota(jnp.int32, (_BLK, G), 1)).astype(jnp.float32)
    ct01 = ct01_ref[...]
    ctf = dinv * (ct01[:, 0:G] + ct01[:, G:2 * G]) + dinv2 * onehot
    dn = (((0,), (0,)), ((), ()))

    @pl.when(i == 0)
    def _():
        accm[...] = jnp.zeros_like(accm)
        accl[...] = jnp.zeros_like(accl)

    accm[...] += lax.dot_general(ctf, hm, dimension_numbers=dn,
                                 preferred_element_type=jnp.float32)
    accl[...] += lax.dot_general(ctf, hl, dimension_numbers=dn,
                                 preferred_element_type=jnp.float32)

    @pl.when(i == _NBLK - 1)
    def _():
        cntinv = cntinv_ref[...]
        zm_ref[...] = jnp.dot(accm[...] * cntinv, w2_ref[...],
                              preferred_element_type=jnp.float32) + b2_ref[...]
        zl_ref[...] = jnp.dot(accl[...] * cntinv, w4_ref[...],
                              preferred_element_type=jnp.float32) + b4_ref[...]


_dense_call = pl.pallas_call(
    _dense_body,
    grid=(_NBLK,),
    in_specs=[
        pl.BlockSpec((_BLK, DIN), lambda i: (i, 0)),
        pl.BlockSpec((_BLK, 128), lambda i: (i, 0)),
        pl.BlockSpec((_BLK, 128), lambda i: (i, 0)),
        pl.BlockSpec((_BLK, 128), lambda i: (i, 0)),
        pl.BlockSpec((_BLK, 1), lambda i: (i, 0)),
        pl.BlockSpec((_BLK, 1), lambda i: (i, 0)),
        pl.BlockSpec((G, 1), lambda i: (0, 0)),
        pl.BlockSpec((DIN, DH), lambda i: (0, 0)),
        pl.BlockSpec((1, DH), lambda i: (0, 0)),
        pl.BlockSpec((DIN, DH), lambda i: (0, 0)),
        pl.BlockSpec((1, DH), lambda i: (0, 0)),
        pl.BlockSpec((DH, DZ), lambda i: (0, 0)),
        pl.BlockSpec((1, DZ), lambda i: (0, 0)),
        pl.BlockSpec((DH, DZ), lambda i: (0, 0)),
        pl.BlockSpec((1, DZ), lambda i: (0, 0)),
    ],
    out_specs=[
        pl.BlockSpec((G, DZ), lambda i: (0, 0)),
        pl.BlockSpec((G, DZ), lambda i: (0, 0)),
    ],
    out_shape=[
        jax.ShapeDtypeStruct((G, DZ), jnp.float32),
        jax.ShapeDtypeStruct((G, DZ), jnp.float32),
    ],
    scratch_shapes=[
        pltpu.VMEM((G, DH), jnp.float32),
        pltpu.VMEM((G, DH), jnp.float32),
    ],
    compiler_params=pltpu.CompilerParams(
        dimension_semantics=("arbitrary",)),
)


def _pad_ids(ids, per_worker, nworkers, pad_base, nch):
    padded = nch * CH
    npad = padded - per_worker
    padv = pad_base + (jnp.arange(npad, dtype=jnp.int32) % 16)
    padv = jnp.broadcast_to(padv, (nworkers, npad))
    return jnp.concatenate([ids.reshape(nworkers, per_worker), padv], axis=1)


def kernel(x, edge_index, batch, W1, b1, W2, b2, W3, b3, W4, b4):
    src = edge_index[0].astype(jnp.int32)
    dst = edge_index[1].astype(jnp.int32)
    batch = batch.astype(jnp.int32)

    dst_deg = _pad_ids(dst, EW, NW, N, NCH_W).reshape(NW, NCH_W, CH)
    srcy = _pad_ids(src, EC, 16, 0, NCH_C)
    dsty = _pad_ids(dst, EC, 16, N, NCH_C).reshape(16, NCH_C, CH)
    sct = _pad_ids(src, EW, NW, N, NCH_W).reshape(2, 16, NCH_W, CH)
    dct = _pad_ids(dst, EW, NW, 0, NCH_W).reshape(2, 16, NCH_W * CH)

    zeros64 = jnp.zeros((ROWS_PER_SUB, G), jnp.float32)
    zeros16 = jnp.zeros((ROWS_PER_SUB, 16), jnp.float32)
    onehot16 = jnp.zeros((CH, 16), jnp.float32).at[:, 0].set(1.0)

    deg_raw = _deg_kernel(dst_deg, onehot16, zeros16)

    xs0, xs1, xs2, xs3, dinv, cntinv = _prep_call(
        deg_raw[:NPAD], deg_raw[NPAD:], x, batch.reshape(N, 1))

    y01p, y23p, ct01p = _scatter_kernel(
        xs0, xs1, xs2, xs3, srcy, dsty, sct[0], sct[1],
        dct[0], dct[1], batch, dinv.reshape(N), zeros64)

    z_mean, z_logvar = _dense_call(
        x, y01p[:N], y23p[:N], ct01p[:N], dinv,
        batch.reshape(N, 1),
        cntinv.reshape(G, 1),
        W1, b1.reshape(1, DH), W3, b3.reshape(1, DH),
        W2, b2.reshape(1, DZ), W4, b4.reshape(1, DZ))
    return (z_mean, z_logvar)
